# baseline (speedup 1.0000x reference)
import jax
import jax.numpy as jnp
from jax import lax
from jax.experimental import pallas as pl
from jax.experimental.pallas import tpu as pltpu
from jax.experimental.pallas import tpu_sc as plsc

V = 100000
D = 300
B = 256

_LENS = (200, 30, 30, 20, 20, 50, 50)
_NW = 32
_SUB = 80
_CHUNKS = tuple(B * L // _NW for L in _LENS)
_TOFF = tuple(sum(_CHUNKS[:t]) for t in range(7))
_TOTAL = sum(_CHUNKS)


def _body(*refs):
    idx_hbm = refs[0:7]
    table_hbm = refs[7]
    outs = refs[8:15]
    idx_v = refs[15]
    bufs = refs[16:18]
    cbufs = refs[18:20]
    sems = refs[20:22]
    sem_idx = refs[22]

    wid = lax.axis_index("s") * 2 + lax.axis_index("c")

    for t in range(7):
        pltpu.async_copy(
            idx_hbm[t].at[pl.ds(wid * _CHUNKS[t], _CHUNKS[t])],
            idx_v.at[pl.ds(_TOFF[t], _CHUNKS[t])], sem_idx)
    pltpu.make_async_copy(
        idx_hbm[0].at[pl.ds(0, _TOTAL)], idx_v, sem_idx).wait()

    def start(t, i, s):
        idx_sl = idx_v.at[pl.ds(_TOFF[t] + i * _SUB, _SUB)]
        pltpu.async_copy(table_hbm.at[idx_sl], bufs[s], sems[s])

    def finish(i, s, out_ref, base):
        buf, cbuf = bufs[s], cbufs[s]
        pltpu.make_async_copy(
            table_hbm.at[idx_v.at[pl.ds(0, _SUB)]], buf, sems[s]).wait()

        def row_step(r, _):
            cbuf[r, pl.ds(0, 16)] = buf[r, pl.ds(256, 16)]
            cbuf[r, pl.ds(16, 16)] = buf[r, pl.ds(272, 16)]
            cbuf[r, pl.ds(28, 16)] = buf[r, pl.ds(284, 16)]
            return 0

        lax.fori_loop(0, _SUB, row_step, 0, unroll=4)

        rows = pl.ds(base + i * _SUB, _SUB)
        pltpu.sync_copy(buf.at[:, pl.ds(0, 256)], out_ref.at[rows, pl.ds(0, 256)])
        pltpu.sync_copy(cbuf, out_ref.at[rows, pl.ds(256, 44)])

    for t in range(7):
        chunk = _CHUNKS[t]
        base = wid * chunk
        n_sub = chunk // _SUB
        out_ref = outs[t]

        start(t, 0, 0)

        def sub_step(i, _, t=t, n_sub=n_sub, out_ref=out_ref, base=base):
            @pl.when(i % 2 == 0)
            def _():
                @pl.when(i + 1 < n_sub)
                def _():
                    start(t, i + 1, 1)
                finish(i, 0, out_ref, base)

            @pl.when(i % 2 == 1)
            def _():
                @pl.when(i + 1 < n_sub)
                def _():
                    start(t, i + 1, 0)
                finish(i, 1, out_ref, base)

            return 0

        lax.fori_loop(0, n_sub, sub_step, 0)


@jax.jit
def _gather_all(table, *idx_flat):
    table_p = jnp.pad(table, ((0, 0), (0, 84)))
    mesh = plsc.VectorSubcoreMesh(core_axis_name="c", subcore_axis_name="s")
    out_type = tuple(
        jax.ShapeDtypeStruct((B * L, D), jnp.float32) for L in _LENS
    )
    k = pl.kernel(
        _body,
        out_type=out_type,
        mesh=mesh,
        scratch_types=[pltpu.VMEM((_TOTAL,), jnp.int32),
                       pltpu.VMEM((_SUB, 384), jnp.float32),
                       pltpu.VMEM((_SUB, 384), jnp.float32),
                       pltpu.VMEM((_SUB, 44), jnp.float32),
                       pltpu.VMEM((_SUB, 44), jnp.float32)]
        + [pltpu.SemaphoreType.DMA] * 3,
    )
    return k(*idx_flat, table_p)


def kernel(passage, passage_mask, question, question_mask, questioninfo,
           questioninfo_mask, answer1, answer1_mask, answer2, answer2_mask,
           qanswer1, qanswer1_mask, qanswer2, qanswer2_mask, table):
    idxs = (passage, question, questioninfo, answer1, answer2, qanswer1,
            qanswer2)
    flat = tuple(a.reshape(-1) for a in idxs)
    embs = _gather_all(table, *flat)
    embs = tuple(e.reshape(a.shape[0], a.shape[1], D)
                 for e, a in zip(embs, idxs))
    return (embs[0], passage_mask, embs[1], question_mask, embs[2],
            questioninfo_mask, embs[3], answer1_mask, embs[4], answer2_mask,
            embs[5], qanswer1_mask, embs[6], qanswer2_mask)


# speedup vs baseline: 1.6873x; 1.6873x over previous
import jax
import jax.numpy as jnp
from jax import lax
from jax.experimental import pallas as pl
from jax.experimental.pallas import tpu as pltpu
from jax.experimental.pallas import tpu_sc as plsc

V = 100000
D = 300
B = 256

_LENS = (200, 30, 30, 20, 20, 50, 50)
_NW = 32
_SUB = 80
_CHUNKS = tuple(B * L // _NW for L in _LENS)
_TOFF = tuple(sum(_CHUNKS[:t]) for t in range(7))
_TOTAL = sum(_CHUNKS)


def _body(*refs):
    idx_hbm = refs[0:7]
    table_hbm = refs[7]
    tail_hbm = refs[8]
    outs = refs[9:16]
    idx_v = refs[16]
    sets = (refs[17:20], refs[20:23])
    sems = (refs[23:25], refs[25:27])
    sem_idx = refs[27]

    wid = lax.axis_index("s") * 2 + lax.axis_index("c")

    for t in range(7):
        pltpu.async_copy(
            idx_hbm[t].at[pl.ds(wid * _CHUNKS[t], _CHUNKS[t])],
            idx_v.at[pl.ds(_TOFF[t], _CHUNKS[t])], sem_idx)
    pltpu.make_async_copy(
        idx_hbm[0].at[pl.ds(0, _TOTAL)], idx_v, sem_idx).wait()

    def start(t, i, s):
        idx_sl = idx_v.at[pl.ds(_TOFF[t] + i * _SUB, _SUB)]
        buf_a, buf_b, _ = sets[s]
        sem_a, sem_b = sems[s]
        pltpu.async_copy(table_hbm.at[idx_sl], buf_a, sem_a)
        pltpu.async_copy(tail_hbm.at[idx_sl], buf_b, sem_b)

    def finish(t, i, s, out_ref, base):
        buf_a, buf_b, buf_c = sets[s]
        sem_a, sem_b = sems[s]
        pltpu.make_async_copy(
            table_hbm.at[idx_v.at[pl.ds(0, _SUB)]], buf_a, sem_a).wait()
        pltpu.make_async_copy(
            tail_hbm.at[idx_v.at[pl.ds(0, _SUB)]], buf_b, sem_b).wait()

        def row_step(r, _):
            buf_c[r, pl.ds(0, 16)] = buf_b[r, pl.ds(0, 16)]
            buf_c[r, pl.ds(16, 16)] = buf_b[r, pl.ds(16, 16)]
            buf_c[r, pl.ds(28, 16)] = buf_b[r, pl.ds(28, 16)]
            return 0

        lax.fori_loop(0, _SUB, row_step, 0, unroll=4)

        rows = pl.ds(base + i * _SUB, _SUB)
        pltpu.sync_copy(buf_a, out_ref.at[rows, pl.ds(0, 256)])
        pltpu.sync_copy(buf_c, out_ref.at[rows, pl.ds(256, 44)])

    for t in range(7):
        chunk = _CHUNKS[t]
        base = wid * chunk
        n_sub = chunk // _SUB
        out_ref = outs[t]

        start(t, 0, 0)

        def sub_step(i, _, t=t, n_sub=n_sub, out_ref=out_ref, base=base):
            @pl.when(i % 2 == 0)
            def _():
                @pl.when(i + 1 < n_sub)
                def _():
                    start(t, i + 1, 1)
                finish(t, i, 0, out_ref, base)

            @pl.when(i % 2 == 1)
            def _():
                @pl.when(i + 1 < n_sub)
                def _():
                    start(t, i + 1, 0)
                finish(t, i, 1, out_ref, base)

            return 0

        lax.fori_loop(0, n_sub, sub_step, 0)


@jax.jit
def _gather_all(table, *idx_flat):
    table_a = table[:, :256]
    tail = jnp.pad(table[:, 256:300], ((0, 0), (0, 84)))
    mesh = plsc.VectorSubcoreMesh(core_axis_name="c", subcore_axis_name="s")
    out_type = tuple(
        jax.ShapeDtypeStruct((B * L, D), jnp.float32) for L in _LENS
    )
    buf_set = [
        pltpu.VMEM((_SUB, 256), jnp.float32),
        pltpu.VMEM((_SUB, 128), jnp.float32),
        pltpu.VMEM((_SUB, 44), jnp.float32),
    ]
    k = pl.kernel(
        _body,
        out_type=out_type,
        mesh=mesh,
        scratch_types=[pltpu.VMEM((_TOTAL,), jnp.int32)] + buf_set + buf_set
        + [pltpu.SemaphoreType.DMA] * 5,
    )
    return k(*idx_flat, table_a, tail)


def kernel(passage, passage_mask, question, question_mask, questioninfo,
           questioninfo_mask, answer1, answer1_mask, answer2, answer2_mask,
           qanswer1, qanswer1_mask, qanswer2, qanswer2_mask, table):
    idxs = (passage, question, questioninfo, answer1, answer2, qanswer1,
            qanswer2)
    flat = tuple(a.reshape(-1) for a in idxs)
    embs = _gather_all(table, *flat)
    embs = tuple(e.reshape(a.shape[0], a.shape[1], D)
                 for e, a in zip(embs, idxs))
    return (embs[0], passage_mask, embs[1], question_mask, embs[2],
            questioninfo_mask, embs[3], answer1_mask, embs[4], answer2_mask,
            embs[5], qanswer1_mask, embs[6], qanswer2_mask)


# speedup vs baseline: 1.6976x; 1.0061x over previous
import jax
import jax.numpy as jnp
from jax import lax
from jax.experimental import pallas as pl
from jax.experimental.pallas import tpu as pltpu
from jax.experimental.pallas import tpu_sc as plsc

V = 100000
D = 300
B = 256

_LENS = (200, 30, 30, 20, 20, 50, 50)
_NW = 32
_SUB = 80

_mesh = plsc.VectorSubcoreMesh(core_axis_name="c", subcore_axis_name="s")


def _make_body(lens):
    chunks = tuple(B * L // _NW for L in lens)
    toff = tuple(sum(chunks[:t]) for t in range(len(lens)))
    total = sum(chunks)
    nt = len(lens)

    def body(*refs):
        idx_hbm = refs[0:nt]
        table_hbm = refs[nt]
        tail_hbm = refs[nt + 1]
        outs = refs[nt + 2:2 * nt + 2]
        idx_v = refs[2 * nt + 2]
        sets = (refs[2 * nt + 3:2 * nt + 6], refs[2 * nt + 6:2 * nt + 9])
        sems = (refs[2 * nt + 9:2 * nt + 11], refs[2 * nt + 11:2 * nt + 13])
        sem_idx = refs[2 * nt + 13]

        wid = lax.axis_index("s") * 2 + lax.axis_index("c")

        for t in range(nt):
            pltpu.async_copy(
                idx_hbm[t].at[pl.ds(wid * chunks[t], chunks[t])],
                idx_v.at[pl.ds(toff[t], chunks[t])], sem_idx)
        pltpu.make_async_copy(
            idx_hbm[0].at[pl.ds(0, total)], idx_v, sem_idx).wait()

        def start(t, i, s):
            idx_sl = idx_v.at[pl.ds(toff[t] + i * _SUB, _SUB)]
            buf_a, buf_b, _ = sets[s]
            sem_a, sem_b = sems[s]
            pltpu.async_copy(table_hbm.at[idx_sl, pl.ds(0, 256)], buf_a, sem_a)
            pltpu.async_copy(tail_hbm.at[idx_sl], buf_b, sem_b)

        def finish(i, s, out_ref, base):
            buf_a, buf_b, buf_c = sets[s]
            sem_a, sem_b = sems[s]
            pltpu.make_async_copy(
                table_hbm.at[idx_v.at[pl.ds(0, _SUB)], pl.ds(0, 256)],
                buf_a, sem_a).wait()
            pltpu.make_async_copy(
                tail_hbm.at[idx_v.at[pl.ds(0, _SUB)]], buf_b, sem_b).wait()

            def row_step(r, _):
                buf_c[r, pl.ds(0, 16)] = buf_b[r, pl.ds(0, 16)]
                buf_c[r, pl.ds(16, 16)] = buf_b[r, pl.ds(16, 16)]
                buf_c[r, pl.ds(28, 16)] = buf_b[r, pl.ds(28, 16)]
                return 0

            lax.fori_loop(0, _SUB, row_step, 0, unroll=4)

            rows = pl.ds(base + i * _SUB, _SUB)
            pltpu.sync_copy(buf_a, out_ref.at[rows, pl.ds(0, 256)])
            pltpu.sync_copy(buf_c, out_ref.at[rows, pl.ds(256, 44)])

        for t in range(nt):
            base = wid * chunks[t]
            n_sub = chunks[t] // _SUB
            out_ref = outs[t]

            start(t, 0, 0)

            def sub_step(i, _, t=t, n_sub=n_sub, out_ref=out_ref, base=base):
                @pl.when(i % 2 == 0)
                def _():
                    @pl.when(i + 1 < n_sub)
                    def _():
                        start(t, i + 1, 1)
                    finish(i, 0, out_ref, base)

                @pl.when(i % 2 == 1)
                def _():
                    @pl.when(i + 1 < n_sub)
                    def _():
                        start(t, i + 1, 0)
                    finish(i, 1, out_ref, base)

                return 0

            lax.fori_loop(0, n_sub, sub_step, 0)

    return body, total


def _call(table, tail, idx_list, lens):
    body, total = _make_body(lens)
    out_type = tuple(
        jax.ShapeDtypeStruct((B * L, D), jnp.float32) for L in lens
    )
    buf_set = [
        pltpu.VMEM((_SUB, 256), jnp.float32),
        pltpu.VMEM((_SUB, 128), jnp.float32),
        pltpu.VMEM((_SUB, 44), jnp.float32),
    ]
    k = pl.kernel(
        body,
        out_type=out_type,
        mesh=_mesh,
        scratch_types=[pltpu.VMEM((total,), jnp.int32)] + buf_set + buf_set
        + [pltpu.SemaphoreType.DMA] * 5,
    )
    return k(*idx_list, table, tail)


@jax.jit
def _gather_all(table, *idx_flat):
    tail = jnp.pad(table[:, 256:300], ((0, 0), (0, 84)))

    outs_b = _call(table, tail, idx_flat[0:1], _LENS[0:1])
    outs_c = _call(table, tail, idx_flat[5:7], _LENS[5:7])
    outs_d = _call(table, tail, idx_flat[1:5], _LENS[1:5])

    return (outs_b[0], outs_d[0], outs_d[1], outs_d[2], outs_d[3],
            outs_c[0], outs_c[1])


def kernel(passage, passage_mask, question, question_mask, questioninfo,
           questioninfo_mask, answer1, answer1_mask, answer2, answer2_mask,
           qanswer1, qanswer1_mask, qanswer2, qanswer2_mask, table):
    idxs = (passage, question, questioninfo, answer1, answer2, qanswer1,
            qanswer2)
    flat = tuple(a.reshape(-1) for a in idxs)
    embs = _gather_all(table, *flat)
    embs = tuple(e.reshape(a.shape[0], a.shape[1], D)
                 for e, a in zip(embs, idxs))
    return (embs[0], passage_mask, embs[1], question_mask, embs[2],
            questioninfo_mask, embs[3], answer1_mask, embs[4], answer2_mask,
            embs[5], qanswer1_mask, embs[6], qanswer2_mask)
